# baseline (speedup 1.0000x reference)
import functools

import jax
import jax.numpy as jnp
from jax import lax
from jax.experimental import pallas as pl
from jax.experimental.pallas import tpu as pltpu
from jax.experimental.pallas import tpu_sc as plsc

N = 100000
E = 1600000
IN_DIM = 32
HID = 32
NL = 3
EMB = HID * NL
G = 256

NC = 2
NS = 16

NHALF = N // NC
TRASH = 512
ACCR = 51200
ZROWS = 64
EPAD = 1638400
ROWS_PER_TILE = EPAD // 128 // NS
CJ = 4
CHUNKS = ROWS_PER_TILE // CJ
STAGE = CJ * 128 + 128
HALF_STRIDE = 52000
AGG_PAD = NC * HALF_STRIDE

_HIGH = jax.lax.Precision.HIGHEST
_SC_MESH = plsc.VectorSubcoreMesh(core_axis_name="c", subcore_axis_name="s")


@functools.partial(
    pl.kernel,
    out_type=jax.ShapeDtypeStruct((AGG_PAD, HID), jnp.float32),
    mesh=_SC_MESH,
    compiler_params=pltpu.CompilerParams(use_tc_tiling_on_sc=False,
                                         needs_layout_passes=False),
    scratch_types=[
        pltpu.VMEM_SHARED((ACCR, HID), jnp.float32),
        pltpu.VMEM((ZROWS, HID), jnp.float32),
        pltpu.VMEM((CJ, 128), jnp.int32),
        pltpu.VMEM((CJ, 128), jnp.int32),
        pltpu.VMEM((STAGE,), jnp.int32),
        pltpu.VMEM((STAGE,), jnp.int32),
        pltpu.VMEM((CJ, 128), jnp.int32),
        pltpu.VMEM((CJ, 128, HID), jnp.float32),
        pltpu.SemaphoreType.DMA,
        pltpu.SemaphoreType.DMA,
    ],
)
def _sc_scatter(x_hbm, src_hbm, dst_hbm, out_hbm,
                acc, zbuf, srcb, dstb, stsrc, stdst, fireb, rowsb, gsem, ssem):
    c = lax.axis_index("c")
    s = lax.axis_index("s")
    lo = c * NHALF

    zeros16 = jnp.zeros((16,), jnp.float32)

    def _zrow(i, carry):
        zbuf[i, pl.ds(0, 16)] = zeros16
        zbuf[i, pl.ds(16, 16)] = zeros16
        return carry

    lax.fori_loop(0, ZROWS, _zrow, 0)
    zn = (ACCR // NS) // ZROWS

    def _zacc(k, carry):
        pltpu.async_copy(
            zbuf, acc.at[pl.ds(s * (ACCR // NS) + k * ZROWS, ZROWS)], gsem)
        return carry

    lax.fori_loop(0, zn, _zacc, 0)

    def _zwait(k, carry):
        pltpu.make_async_copy(
            zbuf, acc.at[pl.ds(s * (ACCR // NS) + k * ZROWS, ZROWS)],
            gsem).wait()
        return carry

    lax.fori_loop(0, zn, _zwait, 0)
    plsc.subcore_barrier()

    def _chunk(g, carry):
        r0 = s * ROWS_PER_TILE + g * CJ
        pltpu.sync_copy(src_hbm.at[pl.ds(r0, CJ)], srcb)
        pltpu.sync_copy(dst_hbm.at[pl.ds(r0, CJ)], dstb)
        ptr = jnp.int32(0)
        for j in range(CJ):
            for k in range(8):
                d = dstb[j, pl.ds(k * 16, 16)]
                sv = srcb[j, pl.ds(k * 16, 16)]
                dl = d - lo
                inr = (dl >= 0) & (dl < NHALF)
                pos = plsc.cumsum(jnp.where(inr, 1, 0)) + (ptr - 1)
                plsc.store_scatter(stsrc, [pos], sv, mask=inr)
                plsc.store_scatter(stdst, [pos], dl, mask=inr)
                ptr = ptr + plsc.all_reduce_population_count(inr)[0]
        pad_src = jnp.zeros((16,), jnp.int32)
        pad_dst = jnp.full((16,), NHALF, jnp.int32)
        for m in range(8):
            stsrc[pl.ds(ptr + m * 16, 16)] = pad_src
            stdst[pl.ds(ptr + m * 16, 16)] = pad_dst
        nops = (ptr + 127) >> 7
        for j in range(CJ):
            @pl.when(j < nops)
            def _():
                pltpu.async_copy(x_hbm.at[stsrc.at[pl.ds(j * 128, 128)]],
                                 rowsb.at[j], gsem)
        for j in range(CJ):
            @pl.when(j < nops)
            def _():
                pltpu.make_async_copy(
                    x_hbm.at[stsrc.at[pl.ds(j * 128, 128)]],
                    rowsb.at[j], gsem).wait()
        for j in range(CJ):
            @pl.when(j < nops)
            def _():
                for m in range(8):
                    fireb[j, pl.ds(m * 16, 16)] = stdst[
                        pl.ds(j * 128 + m * 16, 16)]
                pltpu.async_copy(rowsb.at[j], acc.at[fireb.at[j]], ssem,
                                 add=True)
        for j in range(CJ):
            @pl.when(j < nops)
            def _():
                pltpu.make_async_copy(
                    rowsb.at[j], acc.at[fireb.at[j]], ssem).wait()
        return carry

    lax.fori_loop(0, CHUNKS, _chunk, 0)
    plsc.subcore_barrier()

    wb = ACCR // NS
    pltpu.sync_copy(acc.at[pl.ds(s * wb, wb)],
                    out_hbm.at[pl.ds(c * HALF_STRIDE + s * wb, wb)])


@functools.partial(
    pl.kernel,
    out_type=[jax.ShapeDtypeStruct((G, HID), jnp.float32) for _ in range(NL)],
    mesh=_SC_MESH,
    compiler_params=pltpu.CompilerParams(use_tc_tiling_on_sc=False),
    scratch_types=[
        pltpu.VMEM((16,), jnp.int32),
        pltpu.VMEM((16, HID), jnp.float32),
        pltpu.SemaphoreType.DMA,
    ],
)
def _sc_gather(h0, h1, h2, sel, o0, o1, o2, selb, gbuf, sem):
    c = lax.axis_index("c")
    s = lax.axis_index("s")

    @pl.when(c == 0)
    def _():
        pltpu.sync_copy(sel.at[pl.ds(s * 16, 16)], selb)
        for h, o in ((h0, o0), (h1, o1), (h2, o2)):
            pltpu.async_copy(h.at[selb], gbuf, sem).wait()
            pltpu.sync_copy(gbuf, o.at[pl.ds(s * 16, 16)])


BN_ROWS = 2000
NGRID = N // BN_ROWS


def _row_spec(w):
    return pl.BlockSpec((BN_ROWS, w), lambda i: (i, 0))


_agg_spec = pl.BlockSpec((BN_ROWS, HID),
                         lambda i: (jnp.where(i < 25, i, i + 1), 0))


def _full_spec(h, w):
    return pl.BlockSpec((h, w), lambda i: (0, 0))


def _conv_body(x_ref, a_ref, w1_ref, b1_ref, w2_ref, b2_ref,
               h_ref, s_ref, ss_ref):
    i = pl.program_id(0)
    xin = x_ref[...] + a_ref[...]
    h = jnp.maximum(jnp.dot(xin, w1_ref[...], precision=_HIGH) + b1_ref[...], 0.0)
    h = jnp.dot(h, w2_ref[...], precision=_HIGH) + b2_ref[...]
    h = jnp.maximum(h, 0.0)
    h_ref[...] = h
    bs = jnp.sum(h, axis=0, keepdims=True)
    bss = jnp.sum(h * h, axis=0, keepdims=True)

    @pl.when(i == 0)
    def _():
        s_ref[...] = bs
        ss_ref[...] = bss

    @pl.when(i > 0)
    def _():
        s_ref[...] = s_ref[...] + bs
        ss_ref[...] = ss_ref[...] + bss


def _tc_conv(x, agg, w1, b1, w2, b2):
    return pl.pallas_call(
        _conv_body,
        grid=(NGRID,),
        in_specs=[_row_spec(HID), _agg_spec, _full_spec(HID, HID),
                  _full_spec(1, HID), _full_spec(HID, HID), _full_spec(1, HID)],
        out_specs=[_row_spec(HID), _full_spec(1, HID), _full_spec(1, HID)],
        out_shape=[jax.ShapeDtypeStruct((N, HID), jnp.float32),
                   jax.ShapeDtypeStruct((1, HID), jnp.float32),
                   jax.ShapeDtypeStruct((1, HID), jnp.float32)],
    )(x, agg, w1, b1, w2, b2)


def _norm_body(h_ref, s_ref, ss_ref, g_ref, b_ref, o_ref):
    mu = s_ref[...] * (1.0 / N)
    var = ss_ref[...] * (1.0 / N) - mu * mu
    sc = g_ref[...] * lax.rsqrt(var + 1e-5)
    t = b_ref[...] - mu * sc
    o_ref[...] = h_ref[...] * sc + t


def _tc_norm(h, sums, sumsq, gamma, beta):
    return pl.pallas_call(
        _norm_body,
        grid=(NGRID,),
        in_specs=[_row_spec(HID), _full_spec(1, HID), _full_spec(1, HID),
                  _full_spec(1, HID), _full_spec(1, HID)],
        out_specs=_row_spec(HID),
        out_shape=jax.ShapeDtypeStruct((N, HID), jnp.float32),
    )(h, sums, sumsq, gamma, beta)


def _segmax_body(h0_ref, h1_ref, h2_ref, gid_ref, o_ref):
    i = pl.program_id(0)
    le = jnp.concatenate([h0_ref[...], h1_ref[...], h2_ref[...]], axis=1)
    rmax = jnp.max(le, axis=1, keepdims=True)
    oh = gid_ref[...] == lax.broadcasted_iota(jnp.int32, (1, G), 1)
    vals = jnp.where(oh, rmax, -jnp.inf)
    bmax = jnp.max(vals, axis=0, keepdims=True)

    @pl.when(i == 0)
    def _():
        o_ref[...] = bmax

    @pl.when(i > 0)
    def _():
        o_ref[...] = jnp.maximum(o_ref[...], bmax)


def _tc_segmax(h0, h1, h2, gid2d):
    return pl.pallas_call(
        _segmax_body,
        grid=(NGRID,),
        in_specs=[_row_spec(HID), _row_spec(HID), _row_spec(HID),
                  _row_spec(1)],
        out_specs=_full_spec(1, G),
        out_shape=jax.ShapeDtypeStruct((1, G), jnp.float32),
    )(h0, h1, h2, gid2d)


def _argmin_body(h0_ref, h1_ref, h2_ref, gid_ref, smax_ref, o_ref):
    i = pl.program_id(0)
    le = jnp.concatenate([h0_ref[...], h1_ref[...], h2_ref[...]], axis=1)
    rmax = jnp.max(le, axis=1, keepdims=True)
    oh = gid_ref[...] == lax.broadcasted_iota(jnp.int32, (1, G), 1)
    segv = jnp.sum(jnp.where(oh, smax_ref[...], 0.0), axis=1, keepdims=True)
    ridx = i * BN_ROWS + lax.broadcasted_iota(jnp.int32, (BN_ROWS, 1), 0)
    cand = jnp.where(rmax == segv, ridx, N)
    candm = jnp.where(oh, cand, N)
    bmin = jnp.min(candm, axis=0, keepdims=True)

    @pl.when(i == 0)
    def _():
        o_ref[...] = bmin

    @pl.when(i > 0)
    def _():
        o_ref[...] = jnp.minimum(o_ref[...], bmin)

    @pl.when(i == NGRID - 1)
    def _():
        o_ref[...] = jnp.minimum(o_ref[...], N - 1)


def _tc_argmin(h0, h1, h2, gid2d, smax):
    return pl.pallas_call(
        _argmin_body,
        grid=(NGRID,),
        in_specs=[_row_spec(HID), _row_spec(HID), _row_spec(HID),
                  _row_spec(1), _full_spec(1, G)],
        out_specs=_full_spec(1, G),
        out_shape=jax.ShapeDtypeStruct((1, G), jnp.int32),
    )(h0, h1, h2, gid2d, smax)


def _ffn(x, w1, b1, w2, b2, w3, b3, wj, bj):
    h = jnp.maximum(jnp.dot(x, w1, precision=_HIGH) + b1, 0.0)
    h = jnp.maximum(jnp.dot(h, w2, precision=_HIGH) + b2, 0.0)
    h = jnp.maximum(jnp.dot(h, w3, precision=_HIGH) + b3, 0.0)
    return h + jnp.dot(x, wj, precision=_HIGH) + bj


def _sortffn_body(g0_ref, g1_ref, g2_ref,
                  w1_ref, b1_ref, w2_ref, b2_ref, w3_ref, b3_ref,
                  wj_ref, bj_ref, o_ref):
    v = jnp.concatenate([g0_ref[...], g1_ref[...], g2_ref[...]], axis=1)
    iota = lax.broadcasted_iota(jnp.int32, (1, EMB), 1)
    rank = jnp.zeros((G, EMB), jnp.int32)
    for j in range(EMB):
        cj = v[:, j:j + 1]
        lt = cj < v
        tie = (cj == v) & (j < iota)
        rank = rank + jnp.where(lt | tie, 1, 0)
    s = jnp.zeros((G, EMB), jnp.float32)
    for j in range(EMB):
        s = s + jnp.where(rank[:, j:j + 1] == iota, v[:, j:j + 1], 0.0)
    o_ref[...] = _ffn(s, w1_ref[...], b1_ref[...], w2_ref[...], b2_ref[...],
                      w3_ref[...], b3_ref[...], wj_ref[...], bj_ref[...])


def _tc_sortffn(g0, g1, g2, w1, b1, w2, b2, w3, b3, wj, bj):
    fs = _full_spec
    return pl.pallas_call(
        _sortffn_body,
        grid=(1,),
        in_specs=[fs(G, HID)] * 3 + [fs(EMB, EMB), fs(1, EMB)] * 4,
        out_specs=fs(G, EMB),
        out_shape=jax.ShapeDtypeStruct((G, EMB), jnp.float32),
    )(g0, g1, g2, w1, b1, w2, b2, w3, b3, wj, bj)


def _loss_body(h0_ref, h1_ref, h2_ref, gid_ref, gh_ref,
               w1_ref, b1_ref, w2_ref, b2_ref, w3_ref, b3_ref,
               wj_ref, bj_ref, a_ref, p_ref, q_ref):
    i = pl.program_id(0)
    le = jnp.concatenate([h0_ref[...], h1_ref[...], h2_ref[...]], axis=1)
    lh = _ffn(le, w1_ref[...], b1_ref[...], w2_ref[...], b2_ref[...],
              w3_ref[...], b3_ref[...], wj_ref[...], bj_ref[...])
    res = lax.dot_general(lh, gh_ref[...], (((1,), (1,)), ((), ())),
                          precision=_HIGH)
    sp = jnp.maximum(res, 0.0) + jnp.log1p(jnp.exp(-jnp.abs(res)))
    oh = gid_ref[...] == lax.broadcasted_iota(jnp.int32, (1, G), 1)
    s_all = jnp.sum(sp).reshape(1, 1)
    s_pos = jnp.sum(jnp.where(oh, sp, 0.0)).reshape(1, 1)
    s_neg = jnp.sum(jnp.where(oh, sp - res, 0.0)).reshape(1, 1)

    @pl.when(i == 0)
    def _():
        a_ref[...] = s_all
        p_ref[...] = s_pos
        q_ref[...] = s_neg

    @pl.when(i > 0)
    def _():
        a_ref[...] = a_ref[...] + s_all
        p_ref[...] = p_ref[...] + s_pos
        q_ref[...] = q_ref[...] + s_neg


def _tc_loss(h0, h1, h2, gid2d, gh, w1, b1, w2, b2, w3, b3, wj, bj):
    fs = _full_spec
    return pl.pallas_call(
        _loss_body,
        grid=(NGRID,),
        in_specs=[_row_spec(HID)] * 3 + [_row_spec(1), fs(G, EMB)]
                 + [fs(EMB, EMB), fs(1, EMB)] * 4,
        out_specs=[fs(1, 1)] * 3,
        out_shape=[jax.ShapeDtypeStruct((1, 1), jnp.float32)] * 3,
    )(h0, h1, h2, gid2d, gh, w1, b1, w2, b2, w3, b3, wj, bj)


def kernel(feat, edge_index, graph_id, params):
    src = edge_index[0]
    dst = edge_index[1]
    pad = EPAD - E
    src2d = jnp.concatenate(
        [src, jnp.zeros((pad,), jnp.int32)]).reshape(EPAD // 128, 128)
    dst2d = jnp.concatenate(
        [dst, N + (jnp.arange(pad, dtype=jnp.int32) % TRASH)]
    ).reshape(EPAD // 128, 128)
    gid2d = graph_id.reshape(N, 1)

    def b(v):
        return v.reshape(1, -1)

    x = feat
    hs = []
    for i in range(NL):
        agg = _sc_scatter(x, src2d, dst2d)
        h_raw, sums, sumsq = _tc_conv(
            x, agg, params['conv%d_W1' % i], b(params['conv%d_b1' % i]),
            params['conv%d_W2' % i], b(params['conv%d_b2' % i]))
        x = _tc_norm(h_raw, sums, sumsq,
                     b(params['bn%d_g' % i]), b(params['bn%d_b' % i]))
        hs.append(x)

    h0, h1, h2 = hs
    smax = _tc_segmax(h0, h1, h2, gid2d)
    sel = _tc_argmin(h0, h1, h2, gid2d, smax)
    g0, g1, g2 = _sc_gather(h0, h1, h2, sel.reshape(G))
    gw = [params['g_W1'], b(params['g_b1']), params['g_W2'], b(params['g_b2']),
          params['g_W3'], b(params['g_b3']), params['g_Wj'], b(params['g_bj'])]
    lw = [params['l_W1'], b(params['l_b1']), params['l_W2'], b(params['l_b2']),
          params['l_W3'], b(params['l_b3']), params['l_Wj'], b(params['l_bj'])]
    gh = _tc_sortffn(g0, g1, g2, *gw)
    s_all, s_pos, s_neg = _tc_loss(h0, h1, h2, gid2d, gh, *lw)

    log2 = jnp.float32(jnp.log(2.0))
    e_neg = (s_all[0, 0] - s_pos[0, 0]) / (N * (G - 1)) - log2
    e_pos = log2 - s_neg[0, 0] / N
    return e_neg - e_pos


# speedup vs baseline: 2.6795x; 2.6795x over previous
import functools

import jax
import jax.numpy as jnp
from jax import lax
from jax.experimental import pallas as pl
from jax.experimental.pallas import tpu as pltpu
from jax.experimental.pallas import tpu_sc as plsc

N = 100000
E = 1600000
IN_DIM = 32
HID = 32
NL = 3
EMB = HID * NL
G = 256

NC = 2
NS = 16

NHALF = N // NC
TRASH = 512
ACCR = 51200
ZROWS = 64
EPAD = 1634304
ROWS_PER_TILE = EPAD // 128 // NS
JB = 2
CHUNKS = ROWS_PER_TILE // JB
HALF_STRIDE = 52000
AGG_PAD = NC * HALF_STRIDE

_HIGH = jax.lax.Precision.HIGHEST
_SC_MESH = plsc.VectorSubcoreMesh(core_axis_name="c", subcore_axis_name="s")


@functools.partial(
    pl.kernel,
    out_type=jax.ShapeDtypeStruct((AGG_PAD, HID), jnp.float32),
    mesh=_SC_MESH,
    compiler_params=pltpu.CompilerParams(use_tc_tiling_on_sc=False,
                                         needs_layout_passes=False),
    scratch_types=[
        pltpu.VMEM_SHARED((ACCR, HID), jnp.float32),
        pltpu.VMEM((ZROWS, HID), jnp.float32),
        pltpu.VMEM((3, JB, 128), jnp.int32),
        pltpu.VMEM((3, JB, 128), jnp.int32),
        pltpu.VMEM((3, JB, 128, HID), jnp.float32),
        pltpu.SemaphoreType.DMA,
        pltpu.SemaphoreType.DMA,
    ],
)
def _sc_scatter(x_hbm, src_hbm, dst_hbm, out_hbm,
                acc, zbuf, srcb, dstb, rowsb, gsem, ssem):
    c = lax.axis_index("c")
    s = lax.axis_index("s")
    lo = c * NHALF

    zeros16 = jnp.zeros((16,), jnp.float32)

    def _zrow(i, carry):
        zbuf[i, pl.ds(0, 16)] = zeros16
        zbuf[i, pl.ds(16, 16)] = zeros16
        return carry

    lax.fori_loop(0, ZROWS, _zrow, 0)
    zn = (ACCR // NS) // ZROWS

    def _zacc(k, carry):
        pltpu.async_copy(
            zbuf, acc.at[pl.ds(s * (ACCR // NS) + k * ZROWS, ZROWS)], gsem)
        return carry

    lax.fori_loop(0, zn, _zacc, 0)

    def _zwait(k, carry):
        pltpu.make_async_copy(
            zbuf, acc.at[pl.ds(s * (ACCR // NS) + k * ZROWS, ZROWS)],
            gsem).wait()
        return carry

    lax.fori_loop(0, zn, _zwait, 0)
    plsc.subcore_barrier()

    def _load(g, b):
        r0 = s * ROWS_PER_TILE + g * JB
        pltpu.sync_copy(src_hbm.at[pl.ds(r0, JB)], srcb.at[b])
        pltpu.sync_copy(dst_hbm.at[pl.ds(r0, JB)], dstb.at[b])
        for j in range(JB):
            for k in range(8):
                d = dstb[b, j, pl.ds(k * 16, 16)]
                dl = d - lo
                inr = (dl >= 0) & (dl < NHALF)
                dstb[b, j, pl.ds(k * 16, 16)] = jnp.where(
                    inr, dl, NHALF + (d & (TRASH - 1)))

    def _gfire(b):
        for j in range(JB):
            pltpu.async_copy(x_hbm.at[srcb.at[b, j]], rowsb.at[b, j], gsem)

    def _gwait(b):
        for j in range(JB):
            pltpu.make_async_copy(
                x_hbm.at[srcb.at[b, j]], rowsb.at[b, j], gsem).wait()

    def _sfire(b):
        for j in range(JB):
            pltpu.async_copy(rowsb.at[b, j], acc.at[dstb.at[b, j]], ssem,
                             add=True)

    def _swait(b):
        for j in range(JB):
            pltpu.make_async_copy(
                rowsb.at[b, j], acc.at[dstb.at[b, j]], ssem).wait()

    _load(0, 0)
    _gfire(0)
    _load(1, 1)
    _gfire(1)
    _load(2, 2)
    _gwait(0)
    _sfire(0)
    _gfire(2)

    def _trip(h, carry):
        for i in range(3):
            g = 3 * h + 3 + i
            _swait(i)
            _load(g, i)
            _gwait((i + 1) % 3)
            _sfire((i + 1) % 3)
            _gfire(i)
        return carry

    lax.fori_loop(0, (CHUNKS - 3) // 3, _trip, 0)
    _swait((CHUNKS - 3) % 3)
    _gwait((CHUNKS - 2) % 3)
    _sfire((CHUNKS - 2) % 3)
    _gwait((CHUNKS - 1) % 3)
    _sfire((CHUNKS - 1) % 3)
    _swait((CHUNKS - 2) % 3)
    _swait((CHUNKS - 1) % 3)
    plsc.subcore_barrier()

    wb = ACCR // NS
    pltpu.sync_copy(acc.at[pl.ds(s * wb, wb)],
                    out_hbm.at[pl.ds(c * HALF_STRIDE + s * wb, wb)])


@functools.partial(
    pl.kernel,
    out_type=[jax.ShapeDtypeStruct((G, HID), jnp.float32) for _ in range(NL)],
    mesh=_SC_MESH,
    compiler_params=pltpu.CompilerParams(use_tc_tiling_on_sc=False),
    scratch_types=[
        pltpu.VMEM((16,), jnp.int32),
        pltpu.VMEM((16, HID), jnp.float32),
        pltpu.SemaphoreType.DMA,
    ],
)
def _sc_gather(h0, h1, h2, sel, o0, o1, o2, selb, gbuf, sem):
    c = lax.axis_index("c")
    s = lax.axis_index("s")

    @pl.when(c == 0)
    def _():
        pltpu.sync_copy(sel.at[pl.ds(s * 16, 16)], selb)
        for h, o in ((h0, o0), (h1, o1), (h2, o2)):
            pltpu.async_copy(h.at[selb], gbuf, sem).wait()
            pltpu.sync_copy(gbuf, o.at[pl.ds(s * 16, 16)])


BN_ROWS = 2000
NGRID = N // BN_ROWS


def _row_spec(w):
    return pl.BlockSpec((BN_ROWS, w), lambda i: (i, 0))


_agg_spec = pl.BlockSpec((BN_ROWS, HID),
                         lambda i: (jnp.where(i < 25, i, i + 1), 0))


def _full_spec(h, w):
    return pl.BlockSpec((h, w), lambda i: (0, 0))


def _conv_body(x_ref, a_ref, w1_ref, b1_ref, w2_ref, b2_ref,
               h_ref, s_ref, ss_ref):
    i = pl.program_id(0)
    xin = x_ref[...] + a_ref[...]
    h = jnp.maximum(jnp.dot(xin, w1_ref[...], precision=_HIGH) + b1_ref[...], 0.0)
    h = jnp.dot(h, w2_ref[...], precision=_HIGH) + b2_ref[...]
    h = jnp.maximum(h, 0.0)
    h_ref[...] = h
    bs = jnp.sum(h, axis=0, keepdims=True)
    bss = jnp.sum(h * h, axis=0, keepdims=True)

    @pl.when(i == 0)
    def _():
        s_ref[...] = bs
        ss_ref[...] = bss

    @pl.when(i > 0)
    def _():
        s_ref[...] = s_ref[...] + bs
        ss_ref[...] = ss_ref[...] + bss


def _tc_conv(x, agg, w1, b1, w2, b2):
    return pl.pallas_call(
        _conv_body,
        grid=(NGRID,),
        in_specs=[_row_spec(HID), _agg_spec, _full_spec(HID, HID),
                  _full_spec(1, HID), _full_spec(HID, HID), _full_spec(1, HID)],
        out_specs=[_row_spec(HID), _full_spec(1, HID), _full_spec(1, HID)],
        out_shape=[jax.ShapeDtypeStruct((N, HID), jnp.float32),
                   jax.ShapeDtypeStruct((1, HID), jnp.float32),
                   jax.ShapeDtypeStruct((1, HID), jnp.float32)],
    )(x, agg, w1, b1, w2, b2)


def _norm_body(h_ref, s_ref, ss_ref, g_ref, b_ref, o_ref):
    mu = s_ref[...] * (1.0 / N)
    var = ss_ref[...] * (1.0 / N) - mu * mu
    sc = g_ref[...] * lax.rsqrt(var + 1e-5)
    t = b_ref[...] - mu * sc
    o_ref[...] = h_ref[...] * sc + t


def _tc_norm(h, sums, sumsq, gamma, beta):
    return pl.pallas_call(
        _norm_body,
        grid=(NGRID,),
        in_specs=[_row_spec(HID), _full_spec(1, HID), _full_spec(1, HID),
                  _full_spec(1, HID), _full_spec(1, HID)],
        out_specs=_row_spec(HID),
        out_shape=jax.ShapeDtypeStruct((N, HID), jnp.float32),
    )(h, sums, sumsq, gamma, beta)


def _segmax_body(h0_ref, h1_ref, h2_ref, gid_ref, o_ref):
    i = pl.program_id(0)
    le = jnp.concatenate([h0_ref[...], h1_ref[...], h2_ref[...]], axis=1)
    rmax = jnp.max(le, axis=1, keepdims=True)
    oh = gid_ref[...] == lax.broadcasted_iota(jnp.int32, (1, G), 1)
    vals = jnp.where(oh, rmax, -jnp.inf)
    bmax = jnp.max(vals, axis=0, keepdims=True)

    @pl.when(i == 0)
    def _():
        o_ref[...] = bmax

    @pl.when(i > 0)
    def _():
        o_ref[...] = jnp.maximum(o_ref[...], bmax)


def _tc_segmax(h0, h1, h2, gid2d):
    return pl.pallas_call(
        _segmax_body,
        grid=(NGRID,),
        in_specs=[_row_spec(HID), _row_spec(HID), _row_spec(HID),
                  _row_spec(1)],
        out_specs=_full_spec(1, G),
        out_shape=jax.ShapeDtypeStruct((1, G), jnp.float32),
    )(h0, h1, h2, gid2d)


def _argmin_body(h0_ref, h1_ref, h2_ref, gid_ref, smax_ref, o_ref):
    i = pl.program_id(0)
    le = jnp.concatenate([h0_ref[...], h1_ref[...], h2_ref[...]], axis=1)
    rmax = jnp.max(le, axis=1, keepdims=True)
    oh = gid_ref[...] == lax.broadcasted_iota(jnp.int32, (1, G), 1)
    segv = jnp.sum(jnp.where(oh, smax_ref[...], 0.0), axis=1, keepdims=True)
    ridx = i * BN_ROWS + lax.broadcasted_iota(jnp.int32, (BN_ROWS, 1), 0)
    cand = jnp.where(rmax == segv, ridx, N)
    candm = jnp.where(oh, cand, N)
    bmin = jnp.min(candm, axis=0, keepdims=True)

    @pl.when(i == 0)
    def _():
        o_ref[...] = bmin

    @pl.when(i > 0)
    def _():
        o_ref[...] = jnp.minimum(o_ref[...], bmin)

    @pl.when(i == NGRID - 1)
    def _():
        o_ref[...] = jnp.minimum(o_ref[...], N - 1)


def _tc_argmin(h0, h1, h2, gid2d, smax):
    return pl.pallas_call(
        _argmin_body,
        grid=(NGRID,),
        in_specs=[_row_spec(HID), _row_spec(HID), _row_spec(HID),
                  _row_spec(1), _full_spec(1, G)],
        out_specs=_full_spec(1, G),
        out_shape=jax.ShapeDtypeStruct((1, G), jnp.int32),
    )(h0, h1, h2, gid2d, smax)


def _ffn(x, w1, b1, w2, b2, w3, b3, wj, bj):
    h = jnp.maximum(jnp.dot(x, w1, precision=_HIGH) + b1, 0.0)
    h = jnp.maximum(jnp.dot(h, w2, precision=_HIGH) + b2, 0.0)
    h = jnp.maximum(jnp.dot(h, w3, precision=_HIGH) + b3, 0.0)
    return h + jnp.dot(x, wj, precision=_HIGH) + bj


def _sortffn_body(g0_ref, g1_ref, g2_ref,
                  w1_ref, b1_ref, w2_ref, b2_ref, w3_ref, b3_ref,
                  wj_ref, bj_ref, o_ref):
    v = jnp.concatenate([g0_ref[...], g1_ref[...], g2_ref[...]], axis=1)
    iota = lax.broadcasted_iota(jnp.int32, (1, EMB), 1)
    rank = jnp.zeros((G, EMB), jnp.int32)
    for j in range(EMB):
        cj = v[:, j:j + 1]
        lt = cj < v
        tie = (cj == v) & (j < iota)
        rank = rank + jnp.where(lt | tie, 1, 0)
    s = jnp.zeros((G, EMB), jnp.float32)
    for j in range(EMB):
        s = s + jnp.where(rank[:, j:j + 1] == iota, v[:, j:j + 1], 0.0)
    o_ref[...] = _ffn(s, w1_ref[...], b1_ref[...], w2_ref[...], b2_ref[...],
                      w3_ref[...], b3_ref[...], wj_ref[...], bj_ref[...])


def _tc_sortffn(g0, g1, g2, w1, b1, w2, b2, w3, b3, wj, bj):
    fs = _full_spec
    return pl.pallas_call(
        _sortffn_body,
        grid=(1,),
        in_specs=[fs(G, HID)] * 3 + [fs(EMB, EMB), fs(1, EMB)] * 4,
        out_specs=fs(G, EMB),
        out_shape=jax.ShapeDtypeStruct((G, EMB), jnp.float32),
    )(g0, g1, g2, w1, b1, w2, b2, w3, b3, wj, bj)


def _loss_body(h0_ref, h1_ref, h2_ref, gid_ref, gh_ref,
               w1_ref, b1_ref, w2_ref, b2_ref, w3_ref, b3_ref,
               wj_ref, bj_ref, a_ref, p_ref, q_ref):
    i = pl.program_id(0)
    le = jnp.concatenate([h0_ref[...], h1_ref[...], h2_ref[...]], axis=1)
    lh = _ffn(le, w1_ref[...], b1_ref[...], w2_ref[...], b2_ref[...],
              w3_ref[...], b3_ref[...], wj_ref[...], bj_ref[...])
    res = lax.dot_general(lh, gh_ref[...], (((1,), (1,)), ((), ())),
                          precision=_HIGH)
    sp = jnp.maximum(res, 0.0) + jnp.log1p(jnp.exp(-jnp.abs(res)))
    oh = gid_ref[...] == lax.broadcasted_iota(jnp.int32, (1, G), 1)
    s_all = jnp.sum(sp).reshape(1, 1)
    s_pos = jnp.sum(jnp.where(oh, sp, 0.0)).reshape(1, 1)
    s_neg = jnp.sum(jnp.where(oh, sp - res, 0.0)).reshape(1, 1)

    @pl.when(i == 0)
    def _():
        a_ref[...] = s_all
        p_ref[...] = s_pos
        q_ref[...] = s_neg

    @pl.when(i > 0)
    def _():
        a_ref[...] = a_ref[...] + s_all
        p_ref[...] = p_ref[...] + s_pos
        q_ref[...] = q_ref[...] + s_neg


def _tc_loss(h0, h1, h2, gid2d, gh, w1, b1, w2, b2, w3, b3, wj, bj):
    fs = _full_spec
    return pl.pallas_call(
        _loss_body,
        grid=(NGRID,),
        in_specs=[_row_spec(HID)] * 3 + [_row_spec(1), fs(G, EMB)]
                 + [fs(EMB, EMB), fs(1, EMB)] * 4,
        out_specs=[fs(1, 1)] * 3,
        out_shape=[jax.ShapeDtypeStruct((1, 1), jnp.float32)] * 3,
    )(h0, h1, h2, gid2d, gh, w1, b1, w2, b2, w3, b3, wj, bj)


def kernel(feat, edge_index, graph_id, params):
    src = edge_index[0]
    dst = edge_index[1]
    pad = EPAD - E
    src2d = jnp.concatenate(
        [src, jnp.zeros((pad,), jnp.int32)]).reshape(EPAD // 128, 128)
    dst2d = jnp.concatenate(
        [dst, N + (jnp.arange(pad, dtype=jnp.int32) % TRASH)]
    ).reshape(EPAD // 128, 128)
    gid2d = graph_id.reshape(N, 1)

    def b(v):
        return v.reshape(1, -1)

    x = feat
    hs = []
    for i in range(NL):
        agg = _sc_scatter(x, src2d, dst2d)
        h_raw, sums, sumsq = _tc_conv(
            x, agg, params['conv%d_W1' % i], b(params['conv%d_b1' % i]),
            params['conv%d_W2' % i], b(params['conv%d_b2' % i]))
        x = _tc_norm(h_raw, sums, sumsq,
                     b(params['bn%d_g' % i]), b(params['bn%d_b' % i]))
        hs.append(x)

    h0, h1, h2 = hs
    smax = _tc_segmax(h0, h1, h2, gid2d)
    sel = _tc_argmin(h0, h1, h2, gid2d, smax)
    g0, g1, g2 = _sc_gather(h0, h1, h2, sel.reshape(G))
    gw = [params['g_W1'], b(params['g_b1']), params['g_W2'], b(params['g_b2']),
          params['g_W3'], b(params['g_b3']), params['g_Wj'], b(params['g_bj'])]
    lw = [params['l_W1'], b(params['l_b1']), params['l_W2'], b(params['l_b2']),
          params['l_W3'], b(params['l_b3']), params['l_Wj'], b(params['l_bj'])]
    gh = _tc_sortffn(g0, g1, g2, *gw)
    s_all, s_pos, s_neg = _tc_loss(h0, h1, h2, gid2d, gh, *lw)

    log2 = jnp.float32(jnp.log(2.0))
    e_neg = (s_all[0, 0] - s_pos[0, 0]) / (N * (G - 1)) - log2
    e_pos = log2 - s_neg[0, 0] / N
    return e_neg - e_pos


# speedup vs baseline: 3.0051x; 1.1215x over previous
import functools

import jax
import jax.numpy as jnp
from jax import lax
from jax.experimental import pallas as pl
from jax.experimental.pallas import tpu as pltpu
from jax.experimental.pallas import tpu_sc as plsc

N = 100000
E = 1600000
IN_DIM = 32
HID = 32
NL = 3
EMB = HID * NL
G = 256

NC = 2
NS = 16

NHALF = N // NC
TRASH = 512
ACCR = 51200
ZROWS = 64
EPAD = 1634304
ROWS_PER_TILE = EPAD // 128 // NS
JB = 2
CHUNKS = ROWS_PER_TILE // JB
HALF_STRIDE = 52000
AGG_PAD = NC * HALF_STRIDE

_HIGH = jax.lax.Precision.HIGHEST
_SC_MESH = plsc.VectorSubcoreMesh(core_axis_name="c", subcore_axis_name="s")


@functools.partial(
    pl.kernel,
    out_type=jax.ShapeDtypeStruct((AGG_PAD, HID), jnp.float32),
    mesh=_SC_MESH,
    compiler_params=pltpu.CompilerParams(use_tc_tiling_on_sc=False,
                                         needs_layout_passes=False),
    scratch_types=[
        pltpu.VMEM_SHARED((ACCR, HID), jnp.float32),
        pltpu.VMEM((ZROWS, HID), jnp.float32),
        pltpu.VMEM((3, JB, 128), jnp.int32),
        pltpu.VMEM((3, JB, 128), jnp.int32),
        pltpu.VMEM((3, JB, 128, HID), jnp.float32),
        pltpu.SemaphoreType.DMA,
        pltpu.SemaphoreType.DMA,
        pltpu.SemaphoreType.DMA,
    ],
)
def _sc_scatter(x_hbm, src_hbm, dst_hbm, out_hbm,
                acc, zbuf, srcb, dstb, rowsb, gsem, ssem, isem):
    c = lax.axis_index("c")
    s = lax.axis_index("s")
    lo = c * NHALF

    zeros16 = jnp.zeros((16,), jnp.float32)

    def _zrow(i, carry):
        zbuf[i, pl.ds(0, 16)] = zeros16
        zbuf[i, pl.ds(16, 16)] = zeros16
        return carry

    lax.fori_loop(0, ZROWS, _zrow, 0)
    zn = (ACCR // NS) // ZROWS

    def _zacc(k, carry):
        pltpu.async_copy(
            zbuf, acc.at[pl.ds(s * (ACCR // NS) + k * ZROWS, ZROWS)], gsem)
        return carry

    lax.fori_loop(0, zn, _zacc, 0)

    def _zwait(k, carry):
        pltpu.make_async_copy(
            zbuf, acc.at[pl.ds(s * (ACCR // NS) + k * ZROWS, ZROWS)],
            gsem).wait()
        return carry

    lax.fori_loop(0, zn, _zwait, 0)
    plsc.subcore_barrier()

    def _lfire(g, b):
        r0 = s * ROWS_PER_TILE + g * JB
        pltpu.async_copy(src_hbm.at[pl.ds(r0, JB)], srcb.at[b], isem)
        pltpu.async_copy(dst_hbm.at[pl.ds(r0, JB)], dstb.at[b], isem)

    def _lwait(g, b):
        r0 = s * ROWS_PER_TILE + g * JB
        pltpu.make_async_copy(src_hbm.at[pl.ds(r0, JB)], srcb.at[b],
                              isem).wait()
        pltpu.make_async_copy(dst_hbm.at[pl.ds(r0, JB)], dstb.at[b],
                              isem).wait()

    def _fixup(b):
        for j in range(JB):
            for k in range(8):
                d = dstb[b, j, pl.ds(k * 16, 16)]
                dl = d - lo
                inr = (dl >= 0) & (dl < NHALF)
                dstb[b, j, pl.ds(k * 16, 16)] = jnp.where(
                    inr, dl, NHALF + (d & (TRASH - 1)))

    def _gfire(b):
        for j in range(JB):
            pltpu.async_copy(x_hbm.at[srcb.at[b, j]], rowsb.at[b, j], gsem)

    def _gwait(b):
        for j in range(JB):
            pltpu.make_async_copy(
                x_hbm.at[srcb.at[b, j]], rowsb.at[b, j], gsem).wait()

    def _sfire(b):
        for j in range(JB):
            pltpu.async_copy(rowsb.at[b, j], acc.at[dstb.at[b, j]], ssem,
                             add=True)

    def _swait(b):
        for j in range(JB):
            pltpu.make_async_copy(
                rowsb.at[b, j], acc.at[dstb.at[b, j]], ssem).wait()

    _lfire(0, 0)
    _lfire(1, 1)
    _lwait(0, 0)
    _fixup(0)
    _gfire(0)
    _lwait(1, 1)
    _fixup(1)
    _gfire(1)
    _lfire(2, 2)
    _gwait(0)
    _sfire(0)
    _lwait(2, 2)
    _fixup(2)
    _gfire(2)

    def _trip(h, carry):
        for i in range(3):
            g = 3 * h + 3 + i
            _swait(i)
            _lfire(g, i)
            _gwait((i + 1) % 3)
            _sfire((i + 1) % 3)
            _lwait(g, i)
            _fixup(i)
            _gfire(i)
        return carry

    lax.fori_loop(0, (CHUNKS - 3) // 3, _trip, 0)
    _swait((CHUNKS - 3) % 3)
    _gwait((CHUNKS - 2) % 3)
    _sfire((CHUNKS - 2) % 3)
    _gwait((CHUNKS - 1) % 3)
    _sfire((CHUNKS - 1) % 3)
    _swait((CHUNKS - 2) % 3)
    _swait((CHUNKS - 1) % 3)
    plsc.subcore_barrier()

    wb = ACCR // NS
    pltpu.sync_copy(acc.at[pl.ds(s * wb, wb)],
                    out_hbm.at[pl.ds(c * HALF_STRIDE + s * wb, wb)])


@functools.partial(
    pl.kernel,
    out_type=[jax.ShapeDtypeStruct((G, HID), jnp.float32) for _ in range(NL)],
    mesh=_SC_MESH,
    compiler_params=pltpu.CompilerParams(use_tc_tiling_on_sc=False),
    scratch_types=[
        pltpu.VMEM((16,), jnp.int32),
        pltpu.VMEM((16, HID), jnp.float32),
        pltpu.SemaphoreType.DMA,
    ],
)
def _sc_gather(h0, h1, h2, sel, o0, o1, o2, selb, gbuf, sem):
    c = lax.axis_index("c")
    s = lax.axis_index("s")

    @pl.when(c == 0)
    def _():
        pltpu.sync_copy(sel.at[pl.ds(s * 16, 16)], selb)
        for h, o in ((h0, o0), (h1, o1), (h2, o2)):
            pltpu.async_copy(h.at[selb], gbuf, sem).wait()
            pltpu.sync_copy(gbuf, o.at[pl.ds(s * 16, 16)])


BN_ROWS = 2000
NGRID = N // BN_ROWS


def _row_spec(w):
    return pl.BlockSpec((BN_ROWS, w), lambda i: (i, 0))


_agg_spec = pl.BlockSpec((BN_ROWS, HID),
                         lambda i: (jnp.where(i < 25, i, i + 1), 0))


def _full_spec(h, w):
    return pl.BlockSpec((h, w), lambda i: (0, 0))


def _conv_body(x_ref, a_ref, w1_ref, b1_ref, w2_ref, b2_ref,
               h_ref, s_ref, ss_ref):
    i = pl.program_id(0)
    xin = x_ref[...] + a_ref[...]
    h = jnp.maximum(jnp.dot(xin, w1_ref[...], precision=_HIGH) + b1_ref[...], 0.0)
    h = jnp.dot(h, w2_ref[...], precision=_HIGH) + b2_ref[...]
    h = jnp.maximum(h, 0.0)
    h_ref[...] = h
    bs = jnp.sum(h, axis=0, keepdims=True)
    bss = jnp.sum(h * h, axis=0, keepdims=True)

    @pl.when(i == 0)
    def _():
        s_ref[...] = bs
        ss_ref[...] = bss

    @pl.when(i > 0)
    def _():
        s_ref[...] = s_ref[...] + bs
        ss_ref[...] = ss_ref[...] + bss


def _tc_conv(x, agg, w1, b1, w2, b2):
    return pl.pallas_call(
        _conv_body,
        grid=(NGRID,),
        in_specs=[_row_spec(HID), _agg_spec, _full_spec(HID, HID),
                  _full_spec(1, HID), _full_spec(HID, HID), _full_spec(1, HID)],
        out_specs=[_row_spec(HID), _full_spec(1, HID), _full_spec(1, HID)],
        out_shape=[jax.ShapeDtypeStruct((N, HID), jnp.float32),
                   jax.ShapeDtypeStruct((1, HID), jnp.float32),
                   jax.ShapeDtypeStruct((1, HID), jnp.float32)],
    )(x, agg, w1, b1, w2, b2)


def _norm_body(h_ref, s_ref, ss_ref, g_ref, b_ref, o_ref):
    mu = s_ref[...] * (1.0 / N)
    var = ss_ref[...] * (1.0 / N) - mu * mu
    sc = g_ref[...] * lax.rsqrt(var + 1e-5)
    t = b_ref[...] - mu * sc
    o_ref[...] = h_ref[...] * sc + t


def _tc_norm(h, sums, sumsq, gamma, beta):
    return pl.pallas_call(
        _norm_body,
        grid=(NGRID,),
        in_specs=[_row_spec(HID), _full_spec(1, HID), _full_spec(1, HID),
                  _full_spec(1, HID), _full_spec(1, HID)],
        out_specs=_row_spec(HID),
        out_shape=jax.ShapeDtypeStruct((N, HID), jnp.float32),
    )(h, sums, sumsq, gamma, beta)


def _segmax_body(h0_ref, h1_ref, h2_ref, gid_ref, o_ref):
    i = pl.program_id(0)
    le = jnp.concatenate([h0_ref[...], h1_ref[...], h2_ref[...]], axis=1)
    rmax = jnp.max(le, axis=1, keepdims=True)
    oh = gid_ref[...] == lax.broadcasted_iota(jnp.int32, (1, G), 1)
    vals = jnp.where(oh, rmax, -jnp.inf)
    bmax = jnp.max(vals, axis=0, keepdims=True)

    @pl.when(i == 0)
    def _():
        o_ref[...] = bmax

    @pl.when(i > 0)
    def _():
        o_ref[...] = jnp.maximum(o_ref[...], bmax)


def _tc_segmax(h0, h1, h2, gid2d):
    return pl.pallas_call(
        _segmax_body,
        grid=(NGRID,),
        in_specs=[_row_spec(HID), _row_spec(HID), _row_spec(HID),
                  _row_spec(1)],
        out_specs=_full_spec(1, G),
        out_shape=jax.ShapeDtypeStruct((1, G), jnp.float32),
    )(h0, h1, h2, gid2d)


def _argmin_body(h0_ref, h1_ref, h2_ref, gid_ref, smax_ref, o_ref):
    i = pl.program_id(0)
    le = jnp.concatenate([h0_ref[...], h1_ref[...], h2_ref[...]], axis=1)
    rmax = jnp.max(le, axis=1, keepdims=True)
    oh = gid_ref[...] == lax.broadcasted_iota(jnp.int32, (1, G), 1)
    segv = jnp.sum(jnp.where(oh, smax_ref[...], 0.0), axis=1, keepdims=True)
    ridx = i * BN_ROWS + lax.broadcasted_iota(jnp.int32, (BN_ROWS, 1), 0)
    cand = jnp.where(rmax == segv, ridx, N)
    candm = jnp.where(oh, cand, N)
    bmin = jnp.min(candm, axis=0, keepdims=True)

    @pl.when(i == 0)
    def _():
        o_ref[...] = bmin

    @pl.when(i > 0)
    def _():
        o_ref[...] = jnp.minimum(o_ref[...], bmin)

    @pl.when(i == NGRID - 1)
    def _():
        o_ref[...] = jnp.minimum(o_ref[...], N - 1)


def _tc_argmin(h0, h1, h2, gid2d, smax):
    return pl.pallas_call(
        _argmin_body,
        grid=(NGRID,),
        in_specs=[_row_spec(HID), _row_spec(HID), _row_spec(HID),
                  _row_spec(1), _full_spec(1, G)],
        out_specs=_full_spec(1, G),
        out_shape=jax.ShapeDtypeStruct((1, G), jnp.int32),
    )(h0, h1, h2, gid2d, smax)


def _ffn(x, w1, b1, w2, b2, w3, b3, wj, bj):
    h = jnp.maximum(jnp.dot(x, w1, precision=_HIGH) + b1, 0.0)
    h = jnp.maximum(jnp.dot(h, w2, precision=_HIGH) + b2, 0.0)
    h = jnp.maximum(jnp.dot(h, w3, precision=_HIGH) + b3, 0.0)
    return h + jnp.dot(x, wj, precision=_HIGH) + bj


def _sortffn_body(g0_ref, g1_ref, g2_ref,
                  w1_ref, b1_ref, w2_ref, b2_ref, w3_ref, b3_ref,
                  wj_ref, bj_ref, o_ref):
    v = jnp.concatenate([g0_ref[...], g1_ref[...], g2_ref[...]], axis=1)
    iota = lax.broadcasted_iota(jnp.int32, (1, EMB), 1)
    rank = jnp.zeros((G, EMB), jnp.int32)
    for j in range(EMB):
        cj = v[:, j:j + 1]
        lt = cj < v
        tie = (cj == v) & (j < iota)
        rank = rank + jnp.where(lt | tie, 1, 0)
    s = jnp.zeros((G, EMB), jnp.float32)
    for j in range(EMB):
        s = s + jnp.where(rank[:, j:j + 1] == iota, v[:, j:j + 1], 0.0)
    o_ref[...] = _ffn(s, w1_ref[...], b1_ref[...], w2_ref[...], b2_ref[...],
                      w3_ref[...], b3_ref[...], wj_ref[...], bj_ref[...])


def _tc_sortffn(g0, g1, g2, w1, b1, w2, b2, w3, b3, wj, bj):
    fs = _full_spec
    return pl.pallas_call(
        _sortffn_body,
        grid=(1,),
        in_specs=[fs(G, HID)] * 3 + [fs(EMB, EMB), fs(1, EMB)] * 4,
        out_specs=fs(G, EMB),
        out_shape=jax.ShapeDtypeStruct((G, EMB), jnp.float32),
    )(g0, g1, g2, w1, b1, w2, b2, w3, b3, wj, bj)


def _loss_body(h0_ref, h1_ref, h2_ref, gid_ref, gh_ref,
               w1_ref, b1_ref, w2_ref, b2_ref, w3_ref, b3_ref,
               wj_ref, bj_ref, a_ref, p_ref, q_ref):
    i = pl.program_id(0)
    le = jnp.concatenate([h0_ref[...], h1_ref[...], h2_ref[...]], axis=1)
    lh = _ffn(le, w1_ref[...], b1_ref[...], w2_ref[...], b2_ref[...],
              w3_ref[...], b3_ref[...], wj_ref[...], bj_ref[...])
    res = lax.dot_general(lh, gh_ref[...], (((1,), (1,)), ((), ())),
                          precision=_HIGH)
    sp = jnp.maximum(res, 0.0) + jnp.log1p(jnp.exp(-jnp.abs(res)))
    oh = gid_ref[...] == lax.broadcasted_iota(jnp.int32, (1, G), 1)
    s_all = jnp.sum(sp).reshape(1, 1)
    s_pos = jnp.sum(jnp.where(oh, sp, 0.0)).reshape(1, 1)
    s_neg = jnp.sum(jnp.where(oh, sp - res, 0.0)).reshape(1, 1)

    @pl.when(i == 0)
    def _():
        a_ref[...] = s_all
        p_ref[...] = s_pos
        q_ref[...] = s_neg

    @pl.when(i > 0)
    def _():
        a_ref[...] = a_ref[...] + s_all
        p_ref[...] = p_ref[...] + s_pos
        q_ref[...] = q_ref[...] + s_neg


def _tc_loss(h0, h1, h2, gid2d, gh, w1, b1, w2, b2, w3, b3, wj, bj):
    fs = _full_spec
    return pl.pallas_call(
        _loss_body,
        grid=(NGRID,),
        in_specs=[_row_spec(HID)] * 3 + [_row_spec(1), fs(G, EMB)]
                 + [fs(EMB, EMB), fs(1, EMB)] * 4,
        out_specs=[fs(1, 1)] * 3,
        out_shape=[jax.ShapeDtypeStruct((1, 1), jnp.float32)] * 3,
    )(h0, h1, h2, gid2d, gh, w1, b1, w2, b2, w3, b3, wj, bj)


def kernel(feat, edge_index, graph_id, params):
    src = edge_index[0]
    dst = edge_index[1]
    pad = EPAD - E
    src2d = jnp.concatenate(
        [src, jnp.zeros((pad,), jnp.int32)]).reshape(EPAD // 128, 128)
    dst2d = jnp.concatenate(
        [dst, N + (jnp.arange(pad, dtype=jnp.int32) % TRASH)]
    ).reshape(EPAD // 128, 128)
    gid2d = graph_id.reshape(N, 1)

    def b(v):
        return v.reshape(1, -1)

    x = feat
    hs = []
    for i in range(NL):
        agg = _sc_scatter(x, src2d, dst2d)
        h_raw, sums, sumsq = _tc_conv(
            x, agg, params['conv%d_W1' % i], b(params['conv%d_b1' % i]),
            params['conv%d_W2' % i], b(params['conv%d_b2' % i]))
        x = _tc_norm(h_raw, sums, sumsq,
                     b(params['bn%d_g' % i]), b(params['bn%d_b' % i]))
        hs.append(x)

    h0, h1, h2 = hs
    smax = _tc_segmax(h0, h1, h2, gid2d)
    sel = _tc_argmin(h0, h1, h2, gid2d, smax)
    g0, g1, g2 = _sc_gather(h0, h1, h2, sel.reshape(G))
    gw = [params['g_W1'], b(params['g_b1']), params['g_W2'], b(params['g_b2']),
          params['g_W3'], b(params['g_b3']), params['g_Wj'], b(params['g_bj'])]
    lw = [params['l_W1'], b(params['l_b1']), params['l_W2'], b(params['l_b2']),
          params['l_W3'], b(params['l_b3']), params['l_Wj'], b(params['l_bj'])]
    gh = _tc_sortffn(g0, g1, g2, *gw)
    s_all, s_pos, s_neg = _tc_loss(h0, h1, h2, gid2d, gh, *lw)

    log2 = jnp.float32(jnp.log(2.0))
    e_neg = (s_all[0, 0] - s_pos[0, 0]) / (N * (G - 1)) - log2
    e_pos = log2 - s_neg[0, 0] / N
    return e_neg - e_pos


# speedup vs baseline: 3.2695x; 1.0880x over previous
import functools

import jax
import jax.numpy as jnp
from jax import lax
from jax.experimental import pallas as pl
from jax.experimental.pallas import tpu as pltpu
from jax.experimental.pallas import tpu_sc as plsc

N = 100000
E = 1600000
IN_DIM = 32
HID = 32
NL = 3
EMB = HID * NL
G = 256

NC = 2
NS = 16

NHALF = N // NC
TRASH = 512
ACCR = 51200
ZROWS = 64
EPAD = 1634304
ROWS_PER_TILE = EPAD // 128 // NS
JB = 2
CHUNKS = ROWS_PER_TILE // JB
HALF_STRIDE = 60000
AGG_PAD = NC * HALF_STRIDE

_HIGH = jax.lax.Precision.HIGHEST
_SC_MESH = plsc.VectorSubcoreMesh(core_axis_name="c", subcore_axis_name="s")


@functools.partial(
    pl.kernel,
    out_type=jax.ShapeDtypeStruct((AGG_PAD, HID), jnp.float32),
    mesh=_SC_MESH,
    compiler_params=pltpu.CompilerParams(use_tc_tiling_on_sc=False,
                                         needs_layout_passes=False),
    scratch_types=[
        pltpu.VMEM_SHARED((ACCR, HID), jnp.float32),
        pltpu.VMEM((ZROWS, HID), jnp.float32),
        pltpu.VMEM((3, JB, 128), jnp.int32),
        pltpu.VMEM((3, JB, 128), jnp.int32),
        pltpu.VMEM((3, JB, 128, HID), jnp.float32),
        pltpu.SemaphoreType.DMA,
        pltpu.SemaphoreType.DMA,
        pltpu.SemaphoreType.DMA,
    ],
)
def _sc_scatter(x_hbm, src_hbm, dst_hbm, out_hbm,
                acc, zbuf, srcb, dstb, rowsb, gsem, ssem, isem):
    c = lax.axis_index("c")
    s = lax.axis_index("s")
    lo = c * NHALF

    zeros16 = jnp.zeros((16,), jnp.float32)

    def _zrow(i, carry):
        zbuf[i, pl.ds(0, 16)] = zeros16
        zbuf[i, pl.ds(16, 16)] = zeros16
        return carry

    lax.fori_loop(0, ZROWS, _zrow, 0)
    zn = (ACCR // NS) // ZROWS

    def _zacc(k, carry):
        pltpu.async_copy(
            zbuf, acc.at[pl.ds(s * (ACCR // NS) + k * ZROWS, ZROWS)], gsem)
        return carry

    lax.fori_loop(0, zn, _zacc, 0)

    def _zwait(k, carry):
        pltpu.make_async_copy(
            zbuf, acc.at[pl.ds(s * (ACCR // NS) + k * ZROWS, ZROWS)],
            gsem).wait()
        return carry

    lax.fori_loop(0, zn, _zwait, 0)
    plsc.subcore_barrier()

    def _lfire(g, b):
        r0 = s * ROWS_PER_TILE + g * JB
        pltpu.async_copy(src_hbm.at[pl.ds(r0, JB)], srcb.at[b], isem)
        pltpu.async_copy(dst_hbm.at[pl.ds(r0, JB)], dstb.at[b], isem)

    def _lwait(g, b):
        r0 = s * ROWS_PER_TILE + g * JB
        pltpu.make_async_copy(src_hbm.at[pl.ds(r0, JB)], srcb.at[b],
                              isem).wait()
        pltpu.make_async_copy(dst_hbm.at[pl.ds(r0, JB)], dstb.at[b],
                              isem).wait()

    def _fixup(b):
        for j in range(JB):
            for k in range(8):
                d = dstb[b, j, pl.ds(k * 16, 16)]
                dl = d - lo
                inr = (dl >= 0) & (dl < NHALF)
                dstb[b, j, pl.ds(k * 16, 16)] = jnp.where(
                    inr, dl, NHALF + (d & (TRASH - 1)))

    def _gfire(b):
        for j in range(JB):
            pltpu.async_copy(x_hbm.at[srcb.at[b, j]], rowsb.at[b, j], gsem)

    def _gwait(b):
        for j in range(JB):
            pltpu.make_async_copy(
                x_hbm.at[srcb.at[b, j]], rowsb.at[b, j], gsem).wait()

    def _sfire(b):
        for j in range(JB):
            pltpu.async_copy(rowsb.at[b, j], acc.at[dstb.at[b, j]], ssem,
                             add=True)

    def _swait(b):
        for j in range(JB):
            pltpu.make_async_copy(
                rowsb.at[b, j], acc.at[dstb.at[b, j]], ssem).wait()

    _lfire(0, 0)
    _lfire(1, 1)
    _lwait(0, 0)
    _fixup(0)
    _gfire(0)
    _lwait(1, 1)
    _fixup(1)
    _gfire(1)
    _lfire(2, 2)
    _gwait(0)
    _sfire(0)
    _lwait(2, 2)
    _fixup(2)
    _gfire(2)

    def _trip(h, carry):
        for i in range(3):
            g = 3 * h + 3 + i
            _swait(i)
            _lfire(g, i)
            _gwait((i + 1) % 3)
            _sfire((i + 1) % 3)
            _lwait(g, i)
            _fixup(i)
            _gfire(i)
        return carry

    lax.fori_loop(0, (CHUNKS - 3) // 3, _trip, 0)
    _swait((CHUNKS - 3) % 3)
    _gwait((CHUNKS - 2) % 3)
    _sfire((CHUNKS - 2) % 3)
    _gwait((CHUNKS - 1) % 3)
    _sfire((CHUNKS - 1) % 3)
    _swait((CHUNKS - 2) % 3)
    _swait((CHUNKS - 1) % 3)
    plsc.subcore_barrier()

    wb = ACCR // NS
    pltpu.sync_copy(acc.at[pl.ds(s * wb, wb)],
                    out_hbm.at[pl.ds(c * HALF_STRIDE + s * wb, wb)])


@functools.partial(
    pl.kernel,
    out_type=[jax.ShapeDtypeStruct((G, HID), jnp.float32) for _ in range(NL)],
    mesh=_SC_MESH,
    compiler_params=pltpu.CompilerParams(use_tc_tiling_on_sc=False),
    scratch_types=[
        pltpu.VMEM((16,), jnp.int32),
        pltpu.VMEM((16, HID), jnp.float32),
        pltpu.SemaphoreType.DMA,
    ],
)
def _sc_gather(h0, h1, h2, sel, o0, o1, o2, selb, gbuf, sem):
    c = lax.axis_index("c")
    s = lax.axis_index("s")

    @pl.when(c == 0)
    def _():
        pltpu.sync_copy(sel.at[pl.ds(s * 16, 16)], selb)
        for h, o in ((h0, o0), (h1, o1), (h2, o2)):
            pltpu.async_copy(h.at[selb], gbuf, sem).wait()
            pltpu.sync_copy(gbuf, o.at[pl.ds(s * 16, 16)])


BN_ROWS = 10000
NGRID = N // BN_ROWS


def _row_spec(w):
    return pl.BlockSpec((BN_ROWS, w), lambda i: (i, 0))


_agg_spec = pl.BlockSpec((BN_ROWS, HID),
                         lambda i: (jnp.where(i < 5, i, i + 1), 0))


def _full_spec(h, w):
    return pl.BlockSpec((h, w), lambda i: (0, 0))


def _conv_body(x_ref, a_ref, w1_ref, b1_ref, w2_ref, b2_ref,
               h_ref, s_ref, ss_ref):
    i = pl.program_id(0)
    xin = x_ref[...] + a_ref[...]
    h = jnp.maximum(jnp.dot(xin, w1_ref[...], precision=_HIGH) + b1_ref[...], 0.0)
    h = jnp.dot(h, w2_ref[...], precision=_HIGH) + b2_ref[...]
    h = jnp.maximum(h, 0.0)
    h_ref[...] = h
    bs = jnp.sum(h, axis=0, keepdims=True)
    bss = jnp.sum(h * h, axis=0, keepdims=True)

    @pl.when(i == 0)
    def _():
        s_ref[...] = bs
        ss_ref[...] = bss

    @pl.when(i > 0)
    def _():
        s_ref[...] = s_ref[...] + bs
        ss_ref[...] = ss_ref[...] + bss


def _tc_conv(x, agg, w1, b1, w2, b2):
    return pl.pallas_call(
        _conv_body,
        grid=(NGRID,),
        in_specs=[_row_spec(HID), _agg_spec, _full_spec(HID, HID),
                  _full_spec(1, HID), _full_spec(HID, HID), _full_spec(1, HID)],
        out_specs=[_row_spec(HID), _full_spec(1, HID), _full_spec(1, HID)],
        out_shape=[jax.ShapeDtypeStruct((N, HID), jnp.float32),
                   jax.ShapeDtypeStruct((1, HID), jnp.float32),
                   jax.ShapeDtypeStruct((1, HID), jnp.float32)],
    )(x, agg, w1, b1, w2, b2)


def _norm_body(h_ref, s_ref, ss_ref, g_ref, b_ref, o_ref):
    mu = s_ref[...] * (1.0 / N)
    var = ss_ref[...] * (1.0 / N) - mu * mu
    sc = g_ref[...] * lax.rsqrt(var + 1e-5)
    t = b_ref[...] - mu * sc
    o_ref[...] = h_ref[...] * sc + t


def _tc_norm(h, sums, sumsq, gamma, beta):
    return pl.pallas_call(
        _norm_body,
        grid=(NGRID,),
        in_specs=[_row_spec(HID), _full_spec(1, HID), _full_spec(1, HID),
                  _full_spec(1, HID), _full_spec(1, HID)],
        out_specs=_row_spec(HID),
        out_shape=jax.ShapeDtypeStruct((N, HID), jnp.float32),
    )(h, sums, sumsq, gamma, beta)


def _segmax_body(h0_ref, h1_ref, h2_ref, gid_ref, o_ref):
    i = pl.program_id(0)
    le = jnp.concatenate([h0_ref[...], h1_ref[...], h2_ref[...]], axis=1)
    rmax = jnp.max(le, axis=1, keepdims=True)
    oh = gid_ref[...] == lax.broadcasted_iota(jnp.int32, (1, G), 1)
    vals = jnp.where(oh, rmax, -jnp.inf)
    bmax = jnp.max(vals, axis=0, keepdims=True)

    @pl.when(i == 0)
    def _():
        o_ref[...] = bmax

    @pl.when(i > 0)
    def _():
        o_ref[...] = jnp.maximum(o_ref[...], bmax)


def _tc_segmax(h0, h1, h2, gid2d):
    return pl.pallas_call(
        _segmax_body,
        grid=(NGRID,),
        in_specs=[_row_spec(HID), _row_spec(HID), _row_spec(HID),
                  _row_spec(1)],
        out_specs=_full_spec(1, G),
        out_shape=jax.ShapeDtypeStruct((1, G), jnp.float32),
    )(h0, h1, h2, gid2d)


def _argmin_body(h0_ref, h1_ref, h2_ref, gid_ref, smax_ref, o_ref):
    i = pl.program_id(0)
    le = jnp.concatenate([h0_ref[...], h1_ref[...], h2_ref[...]], axis=1)
    rmax = jnp.max(le, axis=1, keepdims=True)
    oh = gid_ref[...] == lax.broadcasted_iota(jnp.int32, (1, G), 1)
    segv = jnp.sum(jnp.where(oh, smax_ref[...], 0.0), axis=1, keepdims=True)
    ridx = i * BN_ROWS + lax.broadcasted_iota(jnp.int32, (BN_ROWS, 1), 0)
    cand = jnp.where(rmax == segv, ridx, N)
    candm = jnp.where(oh, cand, N)
    bmin = jnp.min(candm, axis=0, keepdims=True)

    @pl.when(i == 0)
    def _():
        o_ref[...] = bmin

    @pl.when(i > 0)
    def _():
        o_ref[...] = jnp.minimum(o_ref[...], bmin)

    @pl.when(i == NGRID - 1)
    def _():
        o_ref[...] = jnp.minimum(o_ref[...], N - 1)


def _tc_argmin(h0, h1, h2, gid2d, smax):
    return pl.pallas_call(
        _argmin_body,
        grid=(NGRID,),
        in_specs=[_row_spec(HID), _row_spec(HID), _row_spec(HID),
                  _row_spec(1), _full_spec(1, G)],
        out_specs=_full_spec(1, G),
        out_shape=jax.ShapeDtypeStruct((1, G), jnp.int32),
    )(h0, h1, h2, gid2d, smax)


def _ffn(x, w1, b1, w2, b2, w3, b3, wj, bj, prec=_HIGH):
    h = jnp.maximum(jnp.dot(x, w1, precision=prec) + b1, 0.0)
    h = jnp.maximum(jnp.dot(h, w2, precision=prec) + b2, 0.0)
    h = jnp.maximum(jnp.dot(h, w3, precision=prec) + b3, 0.0)
    return h + jnp.dot(x, wj, precision=prec) + bj


def _sortffn_body(g0_ref, g1_ref, g2_ref,
                  w1_ref, b1_ref, w2_ref, b2_ref, w3_ref, b3_ref,
                  wj_ref, bj_ref, o_ref):
    v = jnp.concatenate([g0_ref[...], g1_ref[...], g2_ref[...]], axis=1)
    iota = lax.broadcasted_iota(jnp.int32, (1, EMB), 1)
    rank = jnp.zeros((G, EMB), jnp.int32)
    for j in range(EMB):
        cj = v[:, j:j + 1]
        lt = cj < v
        tie = (cj == v) & (j < iota)
        rank = rank + jnp.where(lt | tie, 1, 0)
    s = jnp.zeros((G, EMB), jnp.float32)
    for j in range(EMB):
        s = s + jnp.where(rank[:, j:j + 1] == iota, v[:, j:j + 1], 0.0)
    o_ref[...] = _ffn(s, w1_ref[...], b1_ref[...], w2_ref[...], b2_ref[...],
                      w3_ref[...], b3_ref[...], wj_ref[...], bj_ref[...])


def _tc_sortffn(g0, g1, g2, w1, b1, w2, b2, w3, b3, wj, bj):
    fs = _full_spec
    return pl.pallas_call(
        _sortffn_body,
        grid=(1,),
        in_specs=[fs(G, HID)] * 3 + [fs(EMB, EMB), fs(1, EMB)] * 4,
        out_specs=fs(G, EMB),
        out_shape=jax.ShapeDtypeStruct((G, EMB), jnp.float32),
    )(g0, g1, g2, w1, b1, w2, b2, w3, b3, wj, bj)


def _loss_body(h0_ref, h1_ref, h2_ref, gid_ref, gh_ref,
               w1_ref, b1_ref, w2_ref, b2_ref, w3_ref, b3_ref,
               wj_ref, bj_ref, a_ref, p_ref, q_ref):
    i = pl.program_id(0)
    le = jnp.concatenate([h0_ref[...], h1_ref[...], h2_ref[...]], axis=1)
    lh = _ffn(le, w1_ref[...], b1_ref[...], w2_ref[...], b2_ref[...],
              w3_ref[...], b3_ref[...], wj_ref[...], bj_ref[...],
              prec=lax.Precision.DEFAULT)
    res = lax.dot_general(lh, gh_ref[...], (((1,), (1,)), ((), ())),
                          precision=lax.Precision.DEFAULT)
    sp = jnp.maximum(res, 0.0) + jnp.log1p(jnp.exp(-jnp.abs(res)))
    oh = gid_ref[...] == lax.broadcasted_iota(jnp.int32, (1, G), 1)
    s_all = jnp.sum(sp).reshape(1, 1)
    s_pos = jnp.sum(jnp.where(oh, sp, 0.0)).reshape(1, 1)
    s_neg = jnp.sum(jnp.where(oh, sp - res, 0.0)).reshape(1, 1)

    @pl.when(i == 0)
    def _():
        a_ref[...] = s_all
        p_ref[...] = s_pos
        q_ref[...] = s_neg

    @pl.when(i > 0)
    def _():
        a_ref[...] = a_ref[...] + s_all
        p_ref[...] = p_ref[...] + s_pos
        q_ref[...] = q_ref[...] + s_neg


def _tc_loss(h0, h1, h2, gid2d, gh, w1, b1, w2, b2, w3, b3, wj, bj):
    fs = _full_spec
    return pl.pallas_call(
        _loss_body,
        grid=(NGRID,),
        in_specs=[_row_spec(HID)] * 3 + [_row_spec(1), fs(G, EMB)]
                 + [fs(EMB, EMB), fs(1, EMB)] * 4,
        out_specs=[fs(1, 1)] * 3,
        out_shape=[jax.ShapeDtypeStruct((1, 1), jnp.float32)] * 3,
    )(h0, h1, h2, gid2d, gh, w1, b1, w2, b2, w3, b3, wj, bj)


def kernel(feat, edge_index, graph_id, params):
    src = edge_index[0]
    dst = edge_index[1]
    pad = EPAD - E
    src2d = jnp.concatenate(
        [src, jnp.zeros((pad,), jnp.int32)]).reshape(EPAD // 128, 128)
    dst2d = jnp.concatenate(
        [dst, N + (jnp.arange(pad, dtype=jnp.int32) % TRASH)]
    ).reshape(EPAD // 128, 128)
    gid2d = graph_id.reshape(N, 1)

    def b(v):
        return v.reshape(1, -1)

    x = feat
    hs = []
    for i in range(NL):
        agg = _sc_scatter(x, src2d, dst2d)
        h_raw, sums, sumsq = _tc_conv(
            x, agg, params['conv%d_W1' % i], b(params['conv%d_b1' % i]),
            params['conv%d_W2' % i], b(params['conv%d_b2' % i]))
        x = _tc_norm(h_raw, sums, sumsq,
                     b(params['bn%d_g' % i]), b(params['bn%d_b' % i]))
        hs.append(x)

    h0, h1, h2 = hs
    smax = _tc_segmax(h0, h1, h2, gid2d)
    sel = _tc_argmin(h0, h1, h2, gid2d, smax)
    g0, g1, g2 = _sc_gather(h0, h1, h2, sel.reshape(G))
    gw = [params['g_W1'], b(params['g_b1']), params['g_W2'], b(params['g_b2']),
          params['g_W3'], b(params['g_b3']), params['g_Wj'], b(params['g_bj'])]
    lw = [params['l_W1'], b(params['l_b1']), params['l_W2'], b(params['l_b2']),
          params['l_W3'], b(params['l_b3']), params['l_Wj'], b(params['l_bj'])]
    gh = _tc_sortffn(g0, g1, g2, *gw)
    s_all, s_pos, s_neg = _tc_loss(h0, h1, h2, gid2d, gh, *lw)

    log2 = jnp.float32(jnp.log(2.0))
    e_neg = (s_all[0, 0] - s_pos[0, 0]) / (N * (G - 1)) - log2
    e_pos = log2 - s_neg[0, 0] / N
    return e_neg - e_pos
